# Initial kernel scaffold; baseline (speedup 1.0000x reference)
import jax
import jax.numpy as jnp
from jax.experimental import pallas as pl


def kernel(user, item, user_factors, item_factors):
    raise NotImplementedError("write your pallas kernel here")


# speedup vs baseline: 1.4000x; 1.4000x over previous
import functools

import jax
import jax.numpy as jnp
from jax import lax
from jax.experimental import pallas as pl
from jax.experimental.pallas import tpu as pltpu
from jax.experimental.pallas import tpu_sc as plsc

B = 16384
D = 128
NC = 2
NS = 16
NW = NC * NS
RPW = B // NW
CH = 128
NCH = RPW // CH
NBLK = CH // 16


def _dot_chunk(ubuf, ibuf, outv, xpose, chunk, slot):
  def blk_body(blk, _):
    lane = lax.iota(jnp.int32, 16)
    for t in range(16):
      r = blk * 16 + t
      acc = ubuf[slot, r, pl.ds(0, 16)] * ibuf[slot, r, pl.ds(0, 16)]
      for k in range(1, D // 16):
        acc = acc + (ubuf[slot, r, pl.ds(k * 16, 16)]
                     * ibuf[slot, r, pl.ds(k * 16, 16)])
      xpose[t, pl.ds(0, 16)] = acc
    acc_out = plsc.load_gather(xpose, [lane, jnp.zeros((16,), jnp.int32)])
    for j in range(1, 16):
      acc_out = acc_out + plsc.load_gather(
          xpose, [lane, jnp.full((16,), j, jnp.int32)])
    outv[pl.ds(chunk * CH + blk * 16, 16)] = acc_out
    return 0

  lax.fori_loop(0, NBLK, blk_body, 0)


def _mf_kernel(user_hbm, item_hbm, uf_hbm, if_hbm, out_hbm,
               uidx, iidx, ubuf, ibuf, outv, xpose, sem_x, sem_u, sem_i):
  wid = lax.axis_index("s") * NC + lax.axis_index("c")
  base = wid * RPW

  pltpu.async_copy(user_hbm.at[pl.ds(base, RPW)], uidx, sem_x)
  pltpu.async_copy(item_hbm.at[pl.ds(base, RPW)], iidx, sem_x)
  pltpu.make_async_copy(user_hbm.at[pl.ds(base, RPW)], uidx, sem_x).wait()
  pltpu.make_async_copy(item_hbm.at[pl.ds(base, RPW)], iidx, sem_x).wait()

  def start(j, slot):
    pltpu.async_copy(uf_hbm.at[uidx.at[pl.ds(j * CH, CH)]],
                     ubuf.at[slot], sem_u)
    pltpu.async_copy(if_hbm.at[iidx.at[pl.ds(j * CH, CH)]],
                     ibuf.at[slot], sem_i)

  def drain(j, slot):
    pltpu.make_async_copy(uf_hbm.at[uidx.at[pl.ds(j * CH, CH)]],
                          ubuf.at[slot], sem_u).wait()
    pltpu.make_async_copy(if_hbm.at[iidx.at[pl.ds(j * CH, CH)]],
                          ibuf.at[slot], sem_i).wait()

  start(0, 0)

  def pair_body(p, _):
    for b in range(2):
      j = p * 2 + b
      nxt = j + 1

      @pl.when(nxt < NCH)
      def _():
        start(nxt, (j + 1) % 2)
      drain(j, b)
      _dot_chunk(ubuf, ibuf, outv, xpose, j, b)
    return 0

  lax.fori_loop(0, NCH // 2, pair_body, 0)

  pltpu.sync_copy(outv, out_hbm.at[pl.ds(base, RPW)])


@jax.jit
def kernel(user, item, user_factors, item_factors):
  mesh = plsc.VectorSubcoreMesh(
      core_axis_name="c", subcore_axis_name="s",
      num_cores=NC, num_subcores=NS)
  return pl.kernel(
      _mf_kernel,
      out_type=jax.ShapeDtypeStruct((B,), jnp.float32),
      mesh=mesh,
      compiler_params=pltpu.CompilerParams(needs_layout_passes=False),
      scratch_types=[
          pltpu.VMEM((RPW,), jnp.int32),
          pltpu.VMEM((RPW,), jnp.int32),
          pltpu.VMEM((2, CH, D), jnp.float32),
          pltpu.VMEM((2, CH, D), jnp.float32),
          pltpu.VMEM((RPW,), jnp.float32),
          pltpu.VMEM((16, 17), jnp.float32),
          pltpu.SemaphoreType.DMA,
          pltpu.SemaphoreType.DMA,
          pltpu.SemaphoreType.DMA,
      ],
  )(user, item, user_factors, item_factors)
